# baseline (speedup 1.0000x reference)
import functools

import jax
import jax.numpy as jnp
from jax import lax
from jax.experimental import pallas as pl
from jax.experimental.pallas import tpu as pltpu, tpu_sc as plsc

NUM_CORES = 2
NUM_SUBCORES = 16
NUM_WORKERS = NUM_CORES * NUM_SUBCORES
SPLIT = 104
RPG = 2


@functools.partial(jax.jit, static_argnames=("n_rows", "n_tok", "dim"))
def _sc_gather(table, ids, *, n_rows, n_tok, dim):
    rows_per_w = n_rows // NUM_WORKERS
    n_groups = rows_per_w // RPG
    assert n_groups % 2 == 0
    mesh = plsc.VectorSubcoreMesh(core_axis_name="c", subcore_axis_name="s")

    @functools.partial(
        pl.kernel,
        out_type=jax.ShapeDtypeStruct((n_rows, n_tok, dim), jnp.float32),
        mesh=mesh,
        scratch_types=[
            pltpu.VMEM((rows_per_w, n_tok), jnp.int32),
            pltpu.VMEM((2, RPG, n_tok, dim), jnp.float32),
            pltpu.SemaphoreType.DMA,
            pltpu.SemaphoreType.DMA,
        ],
        compiler_params=pltpu.CompilerParams(use_tc_tiling_on_sc=False),
    )
    def k(table_hbm, idx_hbm, out_hbm, idx_v, rows_v, sem_g, sem_o):
        wid = lax.axis_index("s") * NUM_CORES + lax.axis_index("c")
        w_base = wid * rows_per_w
        pltpu.sync_copy(idx_hbm.at[pl.ds(w_base, rows_per_w)], idx_v)

        def gather_group(g, b):
            copies = []
            for r in range(RPG):
                row = g * RPG + r
                copies.append(
                    pltpu.async_copy(
                        table_hbm.at[idx_v.at[row, pl.ds(0, SPLIT)]],
                        rows_v.at[b, r, pl.ds(0, SPLIT)],
                        sem_g,
                    )
                )
                copies.append(
                    pltpu.async_copy(
                        table_hbm.at[idx_v.at[row, pl.ds(SPLIT, n_tok - SPLIT)]],
                        rows_v.at[b, r, pl.ds(SPLIT, n_tok - SPLIT)],
                        sem_g,
                    )
                )
            for c in copies:
                c.wait()

        def out_start(g, b):
            return pltpu.async_copy(
                rows_v.at[b], out_hbm.at[pl.ds(w_base + g * RPG, RPG)], sem_o
            )

        def out_wait(b):
            pltpu.make_async_copy(
                rows_v.at[b], out_hbm.at[pl.ds(w_base, RPG)], sem_o
            ).wait()

        for b in range(2):
            gather_group(b, b)
            out_start(b, b)

        def body(t, carry):
            for b in range(2):
                g = 2 + 2 * t + b
                out_wait(b)
                gather_group(g, b)
                out_start(g, b)
            return carry

        lax.fori_loop(0, (n_groups - 2) // 2, body, 0)
        for b in range(2):
            out_wait(b)

    return k(table, ids)


def kernel(token_ids, embedding_matrix):
    n_rows, n_tok = token_ids.shape
    dim = embedding_matrix.shape[1]
    ids = token_ids.astype(jnp.int32)
    return _sc_gather(embedding_matrix, ids, n_rows=n_rows, n_tok=n_tok, dim=dim)
